# baseline (speedup 1.0000x reference)
import functools

import jax
import jax.numpy as jnp
from jax import lax
from jax.experimental import pallas as pl
from jax.experimental.pallas import tpu as pltpu
from jax.experimental.pallas import tpu_sc as plsc

_N = 10000
_E = 320000
_D = 128
_NC = 2
_NS = 16
_NW = _NS
_EPW = _E // _NW
_C = 80
_NG = _C // 16
_NCH = _EPW // _C
_SC_CH = 5
_NSUP = _NCH // _SC_CH
_NPAD = 10240
_NH = _NPAD // _NC
_RPT = _NH // _NS
_NDU = _NH // 64
_NDR = 128
_RPTD = _NDR // _NS


def _proj_body(x_ref, wl_ref, bl_ref, wr_ref, br_ref, xl_ref, xr_ref):
    xb = x_ref[...]
    xl_ref[...] = jnp.dot(xb, wl_ref[...], preferred_element_type=jnp.float32) + bl_ref[...]
    xr_ref[...] = jnp.dot(xb, wr_ref[...], preferred_element_type=jnp.float32) + br_ref[...]


def _proj(x, W_l, b_l, W_r, b_r):
    blk = 1000
    return pl.pallas_call(
        _proj_body,
        grid=(_N // blk,),
        in_specs=[
            pl.BlockSpec((blk, _D), lambda i: (i, 0)),
            pl.BlockSpec((_D, _D), lambda i: (0, 0)),
            pl.BlockSpec((1, _D), lambda i: (0, 0)),
            pl.BlockSpec((_D, _D), lambda i: (0, 0)),
            pl.BlockSpec((1, _D), lambda i: (0, 0)),
        ],
        out_specs=[
            pl.BlockSpec((blk, _D), lambda i: (i, 0)),
            pl.BlockSpec((blk, _D), lambda i: (i, 0)),
        ],
        out_shape=[jax.ShapeDtypeStruct((_N, _D), jnp.float32)] * 2,
    )(x, W_l, b_l, W_r, b_r)


def _prep_body(dst_ref, loc_ref, d8_ref, val_ref, m8_ref):
    d = dst_ref[...]
    m8_ref[...] = jnp.bitwise_and(d, 63).astype(jnp.float32)
    for c in range(_NC):
        h = d - c * _NH
        hcl = jnp.clip(h, 0, _NH - 1)
        loc_ref[c] = hcl
        d8_ref[c] = lax.shift_right_logical(hcl, 6)
        val_ref[c] = jnp.clip(1 - jnp.abs(h - hcl), 0, 1).astype(jnp.float32)


def _prep(dst):
    return pl.pallas_call(
        _prep_body,
        grid=(_NW,),
        in_specs=[pl.BlockSpec((1, _NCH, _C), lambda w: (w, 0, 0))],
        out_specs=[
            pl.BlockSpec((_NC, 1, _NCH, _C), lambda w: (0, w, 0, 0)),
            pl.BlockSpec((_NC, 1, _NCH, _C), lambda w: (0, w, 0, 0)),
            pl.BlockSpec((_NC, 1, _NCH, _C), lambda w: (0, w, 0, 0)),
            pl.BlockSpec((1, _NCH, _C), lambda w: (w, 0, 0)),
        ],
        out_shape=[
            jax.ShapeDtypeStruct((_NC, _NW, _NCH, _C), jnp.int32),
            jax.ShapeDtypeStruct((_NC, _NW, _NCH, _C), jnp.int32),
            jax.ShapeDtypeStruct((_NC, _NW, _NCH, _C), jnp.float32),
            jax.ShapeDtypeStruct((_NW, _NCH, _C), jnp.float32),
        ],
    )(dst)


def _bcast(v, k):
    return v.at[jnp.full((16,), k, jnp.int32)].get(mode="promise_in_bounds")


def _lane_sum(v):
    lanes = lax.iota(jnp.int32, 16)
    for k in (1, 2, 4, 8):
        v = v + v.at[lanes ^ k].get(mode="promise_in_bounds")
    return v


_MESH = plsc.VectorSubcoreMesh(
    core_axis_name="c", subcore_axis_name="s", num_cores=_NC, num_subcores=_NS
)


@functools.partial(
    pl.kernel,
    out_type=[
        jax.ShapeDtypeStruct((_NPAD, _D), jnp.float32),
        jax.ShapeDtypeStruct((_NC * _NDR, _D), jnp.float32),
    ],
    mesh=_MESH,
    scratch_types=[
        pltpu.VMEM((_SC_CH, _C), jnp.int32),
        pltpu.VMEM((_SC_CH, _C), jnp.int32),
        pltpu.VMEM((_SC_CH, _C), jnp.int32),
        pltpu.VMEM((_SC_CH, _C), jnp.int32),
        pltpu.VMEM((_SC_CH, _C), jnp.float32),
        pltpu.VMEM((_SC_CH, _C), jnp.float32),
        pltpu.VMEM((_SC_CH, _C), jnp.float32),
        pltpu.VMEM((16,), jnp.float32),
        pltpu.VMEM((_D,), jnp.float32),
        pltpu.VMEM((_D,), jnp.float32),
        pltpu.VMEM((_C, _D), jnp.float32),
        pltpu.VMEM((_C, _D), jnp.float32),
        pltpu.VMEM((_C, _D), jnp.float32),
        pltpu.VMEM((_C, _D), jnp.float32),
        pltpu.VMEM((_C, _D), jnp.float32),
        pltpu.VMEM((_C, _D), jnp.float32),
        pltpu.VMEM_SHARED((_NH, _D), jnp.float32),
        pltpu.VMEM_SHARED((_NDR, _D), jnp.float32),
        pltpu.SemaphoreType.DMA,
        pltpu.SemaphoreType.DMA,
    ],
)
def _sc_agg(xl_hbm, xr_hbm, src_hbm, dst_hbm, dstloc_hbm, dstd8_hbm,
            attr_hbm, m8_hbm, valid_hbm, pat_hbm, w_hbm, att_hbm, zero_hbm,
            out_hbm, outden_hbm,
            src_v, dst_v, dstloc_v, dstd8_v, attr_v, m8_v, valid_v,
            pat_v, w_v, att_v, xlr_v, xrr_v, msgA_v, denA_v, msgB_v, denB_v,
            acc_sh, dens_sh, sem, semsc):
    cid = lax.axis_index("c")
    sid = lax.axis_index("s")

    pltpu.sync_copy(zero_hbm, acc_sh.at[pl.ds(sid * _RPT, _RPT)])
    pltpu.sync_copy(zero_hbm.at[pl.ds(0, _RPTD)],
                    dens_sh.at[pl.ds(sid * _RPTD, _RPTD)])
    pltpu.sync_copy(pat_hbm, pat_v)
    pltpu.sync_copy(w_hbm, w_v)
    pltpu.sync_copy(att_hbm, att_v)
    plsc.subcore_barrier()

    pat = pat_v[...]
    padj = [pat + (8.0 * j) for j in range(8)]

    def fire(c2):
        pltpu.async_copy(xl_hbm.at[src_v.at[c2]], xlr_v, sem)
        pltpu.async_copy(xr_hbm.at[dst_v.at[c2]], xrr_v, sem)

    def drain_gathers():
        pltpu.make_async_copy(xl_hbm.at[pl.ds(0, _C)], xlr_v, sem).wait()
        pltpu.make_async_copy(xl_hbm.at[pl.ds(0, _C)], xrr_v, sem).wait()

    def fire_scatters(c2, msg_b, den_b):
        pltpu.async_copy(msg_b, acc_sh.at[dstloc_v.at[c2]], semsc, add=True)
        pltpu.async_copy(den_b, dens_sh.at[dstd8_v.at[c2]], semsc, add=True)

    def drain_scatters(msg_b, den_b):
        pltpu.make_async_copy(zero_hbm.at[pl.ds(0, _C)], msg_b, semsc).wait()
        pltpu.make_async_copy(zero_hbm.at[pl.ds(0, _C)], den_b, semsc).wait()

    def compute(c2, msg_v, den_v):
        def group_body(g, carry2):
            goff = pl.multiple_of(g * 16, 16)
            pa = attr_v[c2, pl.ds(goff, 16)]
            pv = valid_v[c2, pl.ds(goff, 16)]
            pm = m8_v[c2, pl.ds(goff, 16)]
            for k in range(16):
                e = goff + k
                a = _bcast(pa, k)
                vf = _bcast(pv, k)
                m8 = _bcast(pm, k)
                acc = jnp.zeros((16,), jnp.float32)
                for j in range(8):
                    sl = pl.ds(j * 16, 16)
                    u = xlr_v[e, sl] + xrr_v[e, sl] + a * w_v[sl]
                    lr = jnp.maximum(u, 0.2 * u)
                    acc = acc + att_v[sl] * lr
                ex = jnp.exp(_lane_sum(acc)) * vf
                for j in range(8):
                    sl = pl.ds(j * 16, 16)
                    msg_v[e, sl] = ex * xlr_v[e, sl]
                    ind = jnp.maximum(1.0 - jnp.abs(m8 - padj[j]), 0.0)
                    den_v[e, sl] = ex * ind
            return carry2

        lax.fori_loop(0, _NG, group_body, 0)

    def super_body(s, carry0):
        pltpu.sync_copy(src_hbm.at[sid, s], src_v)
        pltpu.sync_copy(dst_hbm.at[sid, s], dst_v)
        pltpu.sync_copy(dstloc_hbm.at[cid, sid, s], dstloc_v)
        pltpu.sync_copy(dstd8_hbm.at[cid, sid, s], dstd8_v)
        pltpu.sync_copy(attr_hbm.at[sid, s], attr_v)
        pltpu.sync_copy(m8_hbm.at[sid, s], m8_v)
        pltpu.sync_copy(valid_hbm.at[cid, sid, s], valid_v)
        fire(0)

        def pair_body(p, carry):
            cA = 2 * p
            drain_gathers()

            @pl.when(p > 0)
            def _dA():
                drain_scatters(msgA_v, denA_v)

            compute(cA, msgA_v, denA_v)
            fire(cA + 1)
            fire_scatters(cA, msgA_v, denA_v)
            drain_gathers()

            @pl.when(p > 0)
            def _dB():
                drain_scatters(msgB_v, denB_v)

            compute(cA + 1, msgB_v, denB_v)
            fire(cA + 2)
            fire_scatters(cA + 1, msgB_v, denB_v)
            return carry

        lax.fori_loop(0, (_SC_CH - 1) // 2, pair_body, 0)
        drain_gathers()
        drain_scatters(msgA_v, denA_v)
        compute(_SC_CH - 1, msgA_v, denA_v)
        fire_scatters(_SC_CH - 1, msgA_v, denA_v)
        drain_scatters(msgA_v, denA_v)
        drain_scatters(msgB_v, denB_v)
        return carry0

    lax.fori_loop(0, _NSUP, super_body, 0)
    plsc.subcore_barrier()
    pltpu.sync_copy(acc_sh.at[pl.ds(sid * _RPT, _RPT)],
                    out_hbm.at[pl.ds(cid * _NH + sid * _RPT, _RPT)])
    pltpu.sync_copy(dens_sh.at[pl.ds(sid * _RPTD, _RPTD)],
                    outden_hbm.at[pl.ds(cid * _NDR + sid * _RPTD, _RPTD)])


def _out_body(a0_ref, d0_ref, bias_ref, wc_ref, bc_ref, o_ref):
    num = a0_ref[...]
    den = d0_ref[...][:, :1]
    spat = num / (den + 1e-16) + bias_ref[...]
    o_ref[...] = jnp.clip(
        jnp.dot(spat, wc_ref[...], preferred_element_type=jnp.float32) + bc_ref[...],
        0.0, 6.0)


def _final(a0, d0, bias, W_c, b_c):
    blk = 400
    return pl.pallas_call(
        _out_body,
        grid=(_N // blk,),
        in_specs=[
            pl.BlockSpec((blk, _D), lambda i: (i, 0)),
            pl.BlockSpec((blk, 2), lambda i: (i, 0)),
            pl.BlockSpec((1, _D), lambda i: (0, 0)),
            pl.BlockSpec((_D, _D), lambda i: (0, 0)),
            pl.BlockSpec((1, _D), lambda i: (0, 0)),
        ],
        out_specs=pl.BlockSpec((blk, _D), lambda i: (i, 0)),
        out_shape=jax.ShapeDtypeStruct((_N, _D), jnp.float32),
    )(a0, d0, bias, W_c, b_c)


def kernel(x, edge_index, edge_attr, W_l, b_l, W_r, b_r, W_e, att, bias, W_c, b_c):
    xl, xr = _proj(x, W_l, b_l.reshape(1, _D), W_r, b_r.reshape(1, _D))
    src = edge_index[0].reshape(_NW, _NCH, _C)
    dst = edge_index[1].reshape(_NW, _NCH, _C)
    dstloc, dstd8, validf, m8 = _prep(dst)
    s4 = (_NW, _NSUP, _SC_CH, _C)
    s5 = (_NC, _NW, _NSUP, _SC_CH, _C)
    ea = edge_attr.reshape(s4)
    zeros = jnp.zeros((_RPT, _D), jnp.float32)
    pat = jnp.repeat(jnp.arange(8, dtype=jnp.float32), 2)
    accm, accd = _sc_agg(xl, xr, src.reshape(s4), dst.reshape(s4),
                         dstloc.reshape(s5), dstd8.reshape(s5), ea,
                         m8.reshape(s4), validf.reshape(s5), pat,
                         W_e[0], att, zeros)
    den = accd.reshape(_NC, _NDR, _D)[:, :_NDU, :].reshape(_NPAD, 2)
    return _final(accm, den, bias.reshape(1, _D), W_c, b_c.reshape(1, _D))


# speedup vs baseline: 1.1251x; 1.1251x over previous
import functools

import jax
import jax.numpy as jnp
from jax import lax
from jax.experimental import pallas as pl
from jax.experimental.pallas import tpu as pltpu
from jax.experimental.pallas import tpu_sc as plsc

_N = 10000
_E = 320000
_D = 128
_NC = 2
_NS = 16
_NW = _NS
_EPW = _E // _NW
_C = 80
_NG = _C // 16
_NCH = _EPW // _C
_SC_CH = 5
_NSUP = _NCH // _SC_CH
_NPAD = 10240
_NH = _NPAD // _NC
_RPT = _NH // _NS
_NDU = _NH // 64
_NDR = 128
_RPTD = _NDR // _NS


def _proj_body(x_ref, wl_ref, bl_ref, wr_ref, br_ref, xl_ref, xr_ref):
    xb = x_ref[...]
    xl_ref[...] = jnp.dot(xb, wl_ref[...], preferred_element_type=jnp.float32) + bl_ref[...]
    xr_ref[...] = jnp.dot(xb, wr_ref[...], preferred_element_type=jnp.float32) + br_ref[...]


def _proj(x, W_l, b_l, W_r, b_r):
    blk = 1000
    return pl.pallas_call(
        _proj_body,
        grid=(_N // blk,),
        in_specs=[
            pl.BlockSpec((blk, _D), lambda i: (i, 0)),
            pl.BlockSpec((_D, _D), lambda i: (0, 0)),
            pl.BlockSpec((1, _D), lambda i: (0, 0)),
            pl.BlockSpec((_D, _D), lambda i: (0, 0)),
            pl.BlockSpec((1, _D), lambda i: (0, 0)),
        ],
        out_specs=[
            pl.BlockSpec((blk, _D), lambda i: (i, 0)),
            pl.BlockSpec((blk, _D), lambda i: (i, 0)),
        ],
        out_shape=[jax.ShapeDtypeStruct((_N, _D), jnp.float32)] * 2,
    )(x, W_l, b_l, W_r, b_r)


def _prep_body(dst_ref, loc_ref, d8_ref, val_ref, m8_ref):
    d = dst_ref[...]
    m8_ref[...] = jnp.bitwise_and(d, 63).astype(jnp.float32)
    for c in range(_NC):
        h = d - c * _NH
        hcl = jnp.clip(h, 0, _NH - 1)
        loc_ref[c] = hcl
        d8_ref[c] = lax.shift_right_logical(hcl, 6)
        val_ref[c] = jnp.clip(1 - jnp.abs(h - hcl), 0, 1).astype(jnp.float32)


def _prep(dst):
    return pl.pallas_call(
        _prep_body,
        grid=(_NW,),
        in_specs=[pl.BlockSpec((1, _NCH, _C), lambda w: (w, 0, 0))],
        out_specs=[
            pl.BlockSpec((_NC, 1, _NCH, _C), lambda w: (0, w, 0, 0)),
            pl.BlockSpec((_NC, 1, _NCH, _C), lambda w: (0, w, 0, 0)),
            pl.BlockSpec((_NC, 1, _NCH, _C), lambda w: (0, w, 0, 0)),
            pl.BlockSpec((1, _NCH, _C), lambda w: (w, 0, 0)),
        ],
        out_shape=[
            jax.ShapeDtypeStruct((_NC, _NW, _NCH, _C), jnp.int32),
            jax.ShapeDtypeStruct((_NC, _NW, _NCH, _C), jnp.int32),
            jax.ShapeDtypeStruct((_NC, _NW, _NCH, _C), jnp.float32),
            jax.ShapeDtypeStruct((_NW, _NCH, _C), jnp.float32),
        ],
    )(dst)


def _bcast(v, k):
    return v.at[jnp.full((16,), k, jnp.int32)].get(mode="promise_in_bounds")


def _lane_sum(v):
    lanes = lax.iota(jnp.int32, 16)
    for k in (1, 2, 4, 8):
        v = v + v.at[lanes ^ k].get(mode="promise_in_bounds")
    return v


_MESH = plsc.VectorSubcoreMesh(
    core_axis_name="c", subcore_axis_name="s", num_cores=_NC, num_subcores=_NS
)


@functools.partial(
    pl.kernel,
    out_type=[
        jax.ShapeDtypeStruct((_NPAD, _D), jnp.float32),
        jax.ShapeDtypeStruct((_NC * _NDR, _D), jnp.float32),
    ],
    mesh=_MESH,
    scratch_types=[
        pltpu.VMEM((_SC_CH, _C), jnp.int32),
        pltpu.VMEM((_SC_CH, _C), jnp.int32),
        pltpu.VMEM((_SC_CH, _C), jnp.int32),
        pltpu.VMEM((_SC_CH, _C), jnp.int32),
        pltpu.VMEM((_SC_CH, _C), jnp.float32),
        pltpu.VMEM((_SC_CH, _C), jnp.float32),
        pltpu.VMEM((_SC_CH, _C), jnp.float32),
        pltpu.VMEM((16,), jnp.float32),
        pltpu.VMEM((_D,), jnp.float32),
        pltpu.VMEM((_D,), jnp.float32),
        pltpu.VMEM((_C, _D), jnp.float32),
        pltpu.VMEM((_C, _D), jnp.float32),
        pltpu.VMEM((_C, _D), jnp.float32),
        pltpu.VMEM((_C, _D), jnp.float32),
        pltpu.VMEM((_C, _D), jnp.float32),
        pltpu.VMEM((_C, _D), jnp.float32),
        pltpu.VMEM_SHARED((_NH, _D), jnp.float32),
        pltpu.VMEM_SHARED((_NDR, _D), jnp.float32),
        pltpu.SemaphoreType.DMA,
        pltpu.SemaphoreType.DMA,
    ],
)
def _sc_agg(xl_hbm, xr_hbm, src_hbm, dst_hbm, dstloc_hbm, dstd8_hbm,
            attr_hbm, m8_hbm, valid_hbm, pat_hbm, w_hbm, att_hbm, zero_hbm,
            out_hbm, outden_hbm,
            src_v, dst_v, dstloc_v, dstd8_v, attr_v, m8_v, valid_v,
            pat_v, w_v, att_v, xlr_v, xrr_v, xlr2_v, xrr2_v, msg_v, den_v,
            acc_sh, dens_sh, sem, sem2):
    cid = lax.axis_index("c")
    sid = lax.axis_index("s")

    pltpu.sync_copy(zero_hbm, acc_sh.at[pl.ds(sid * _RPT, _RPT)])
    pltpu.sync_copy(zero_hbm.at[pl.ds(0, _RPTD)],
                    dens_sh.at[pl.ds(sid * _RPTD, _RPTD)])
    pltpu.sync_copy(pat_hbm, pat_v)
    pltpu.sync_copy(w_hbm, w_v)
    pltpu.sync_copy(att_hbm, att_v)
    plsc.subcore_barrier()

    pat = pat_v[...]
    padj = [pat + (8.0 * j) for j in range(8)]
    ws = [w_v[pl.ds(16 * j, 16)] for j in range(8)]
    ats = [att_v[pl.ds(16 * j, 16)] for j in range(8)]

    def fire(c2, xl_b, xr_b, semx):
        pltpu.async_copy(xl_hbm.at[src_v.at[c2]], xl_b, semx)
        pltpu.async_copy(xr_hbm.at[dst_v.at[c2]], xr_b, semx)

    def drain(xl_b, xr_b, semx):
        pltpu.make_async_copy(xl_hbm.at[pl.ds(0, _C)], xl_b, semx).wait()
        pltpu.make_async_copy(xl_hbm.at[pl.ds(0, _C)], xr_b, semx).wait()

    def compute(c2, xl_b, xr_b):
        def group_body(g, carry2):
            goff = pl.multiple_of(g * 16, 16)
            pa = attr_v[c2, pl.ds(goff, 16)]
            pv = valid_v[c2, pl.ds(goff, 16)]
            pm = m8_v[c2, pl.ds(goff, 16)]
            for k in range(16):
                e = goff + k
                a = _bcast(pa, k)
                vf = _bcast(pv, k)
                m8 = _bcast(pm, k)
                acc = jnp.zeros((16,), jnp.float32)
                xls = []
                for j in range(8):
                    sl = pl.ds(j * 16, 16)
                    xlj = xl_b[e, sl]
                    xls.append(xlj)
                    u = xlj + xr_b[e, sl] + a * ws[j]
                    lr = jnp.maximum(u, 0.2 * u)
                    acc = acc + ats[j] * lr
                ex = jnp.exp(_lane_sum(acc)) * vf
                for j in range(8):
                    sl = pl.ds(j * 16, 16)
                    msg_v[e, sl] = ex * xls[j]
                    ind = jnp.maximum(1.0 - jnp.abs(m8 - padj[j]), 0.0)
                    den_v[e, sl] = ex * ind
            return carry2

        lax.fori_loop(0, _NG, group_body, 0)
        pltpu.sync_copy(msg_v, acc_sh.at[dstloc_v.at[c2]], add=True)
        pltpu.sync_copy(den_v, dens_sh.at[dstd8_v.at[c2]], add=True)

    def super_body(s, carry0):
        pltpu.sync_copy(src_hbm.at[sid, s], src_v)
        pltpu.sync_copy(dst_hbm.at[sid, s], dst_v)
        pltpu.sync_copy(dstloc_hbm.at[cid, sid, s], dstloc_v)
        pltpu.sync_copy(dstd8_hbm.at[cid, sid, s], dstd8_v)
        pltpu.sync_copy(attr_hbm.at[sid, s], attr_v)
        pltpu.sync_copy(m8_hbm.at[sid, s], m8_v)
        pltpu.sync_copy(valid_hbm.at[cid, sid, s], valid_v)
        fire(0, xlr_v, xrr_v, sem)

        def pair_body(p, carry):
            cA = 2 * p
            fire(cA + 1, xlr2_v, xrr2_v, sem2)
            drain(xlr_v, xrr_v, sem)
            compute(cA, xlr_v, xrr_v)
            fire(cA + 2, xlr_v, xrr_v, sem)
            drain(xlr2_v, xrr2_v, sem2)
            compute(cA + 1, xlr2_v, xrr2_v)
            return carry

        lax.fori_loop(0, (_SC_CH - 1) // 2, pair_body, 0)
        drain(xlr_v, xrr_v, sem)
        compute(_SC_CH - 1, xlr_v, xrr_v)
        return carry0

    lax.fori_loop(0, _NSUP, super_body, 0)
    plsc.subcore_barrier()
    pltpu.sync_copy(acc_sh.at[pl.ds(sid * _RPT, _RPT)],
                    out_hbm.at[pl.ds(cid * _NH + sid * _RPT, _RPT)])
    pltpu.sync_copy(dens_sh.at[pl.ds(sid * _RPTD, _RPTD)],
                    outden_hbm.at[pl.ds(cid * _NDR + sid * _RPTD, _RPTD)])


def _out_body(a0_ref, d0_ref, bias_ref, wc_ref, bc_ref, o_ref):
    num = a0_ref[...]
    den = d0_ref[...][:, :1]
    spat = num / (den + 1e-16) + bias_ref[...]
    o_ref[...] = jnp.clip(
        jnp.dot(spat, wc_ref[...], preferred_element_type=jnp.float32) + bc_ref[...],
        0.0, 6.0)


def _final(a0, d0, bias, W_c, b_c):
    blk = 400
    return pl.pallas_call(
        _out_body,
        grid=(_N // blk,),
        in_specs=[
            pl.BlockSpec((blk, _D), lambda i: (i, 0)),
            pl.BlockSpec((blk, 2), lambda i: (i, 0)),
            pl.BlockSpec((1, _D), lambda i: (0, 0)),
            pl.BlockSpec((_D, _D), lambda i: (0, 0)),
            pl.BlockSpec((1, _D), lambda i: (0, 0)),
        ],
        out_specs=pl.BlockSpec((blk, _D), lambda i: (i, 0)),
        out_shape=jax.ShapeDtypeStruct((_N, _D), jnp.float32),
    )(a0, d0, bias, W_c, b_c)


def kernel(x, edge_index, edge_attr, W_l, b_l, W_r, b_r, W_e, att, bias, W_c, b_c):
    xl, xr = _proj(x, W_l, b_l.reshape(1, _D), W_r, b_r.reshape(1, _D))
    src = edge_index[0].reshape(_NW, _NCH, _C)
    dst = edge_index[1].reshape(_NW, _NCH, _C)
    dstloc, dstd8, validf, m8 = _prep(dst)
    s4 = (_NW, _NSUP, _SC_CH, _C)
    s5 = (_NC, _NW, _NSUP, _SC_CH, _C)
    ea = edge_attr.reshape(s4)
    zeros = jnp.zeros((_RPT, _D), jnp.float32)
    pat = jnp.repeat(jnp.arange(8, dtype=jnp.float32), 2)
    accm, accd = _sc_agg(xl, xr, src.reshape(s4), dst.reshape(s4),
                         dstloc.reshape(s5), dstd8.reshape(s5), ea,
                         m8.reshape(s4), validf.reshape(s5), pat,
                         W_e[0], att, zeros)
    den = accd.reshape(_NC, _NDR, _D)[:, :_NDU, :].reshape(_NPAD, 2)
    return _final(accm, den, bias.reshape(1, _D), W_c, b_c.reshape(1, _D))
